# Initial kernel scaffold; baseline (speedup 1.0000x reference)
import jax
import jax.numpy as jnp
from jax.experimental import pallas as pl


def kernel(pos1, pos2, feat1, feat2, W0, g0, b0, W1, g1, b1, W2, g2, b2):
    raise NotImplementedError("write your pallas kernel here")


# speedup vs baseline: 5.2507x; 5.2507x over previous
import functools

import jax
import jax.numpy as jnp
from jax import lax
from jax.experimental import pallas as pl
from jax.experimental.pallas import tpu as pltpu
from jax.experimental.pallas import tpu_sc as plsc

N = 8192
K = 16
C2 = 256
F1 = 64
D = 256
NB = N * K
EPS = 1e-5

QB = 128
MB = 128
RB = 2048


def _knn_body(p1_ref, p2t_ref, idx_ref):
    q = p1_ref[...]
    keys = p2t_ref[...]
    qn = jnp.sum(q * q, axis=1, keepdims=True)
    kn = jnp.sum(keys * keys, axis=0, keepdims=True)
    dot = lax.dot_general(q, keys, (((1,), (0,)), ((), ())),
                          preferred_element_type=jnp.float32)
    s = 2.0 * dot - qn - kn
    iota = lax.broadcasted_iota(jnp.int32, (QB, N), 1)
    cols = []
    for _ in range(K):
        m = jnp.max(s, axis=1, keepdims=True)
        ij = jnp.min(jnp.where(s >= m, iota, N), axis=1)
        cols.append(ij.reshape(QB, 1))
        s = jnp.where(iota == ij[:, None], -jnp.inf, s)
    idx_ref[...] = jnp.concatenate(cols, axis=1)


def _knn(pos1p, pos2t):
    return pl.pallas_call(
        _knn_body,
        grid=(N // QB,),
        in_specs=[
            pl.BlockSpec((QB, 8), lambda i: (i, 0)),
            pl.BlockSpec((8, N), lambda i: (0, 0)),
        ],
        out_specs=pl.BlockSpec((QB, K), lambda i: (i, 0)),
        out_shape=jax.ShapeDtypeStruct((N, K), jnp.int32),
    )(pos1p, pos2t)


def _pre_body(p1_ref, p2_ref, f2_ref, w0c_ref, w0f2_ref, g_ref, p_ref):
    w0c = w0c_ref[...]
    g_ref[...] = (
        lax.dot_general(p2_ref[...], w0c, (((1,), (0,)), ((), ())),
                        preferred_element_type=jnp.float32)
        + lax.dot_general(f2_ref[...], w0f2_ref[...], (((1,), (0,)), ((), ())),
                          preferred_element_type=jnp.float32))
    p_ref[...] = lax.dot_general(p1_ref[...], w0c, (((1,), (0,)), ((), ())),
                                 preferred_element_type=jnp.float32)


def _pre(pos1p, pos2p, feat2, w0cp, w0f2):
    nb = N // 1024
    return pl.pallas_call(
        _pre_body,
        grid=(nb,),
        in_specs=[
            pl.BlockSpec((1024, 8), lambda i: (i, 0)),
            pl.BlockSpec((1024, 8), lambda i: (i, 0)),
            pl.BlockSpec((1024, C2), lambda i: (i, 0)),
            pl.BlockSpec((8, D), lambda i: (0, 0)),
            pl.BlockSpec((C2, D), lambda i: (0, 0)),
        ],
        out_specs=[
            pl.BlockSpec((1024, D), lambda i: (i, 0)),
            pl.BlockSpec((1024, D), lambda i: (i, 0)),
        ],
        out_shape=[
            jax.ShapeDtypeStruct((N, D), jnp.float32),
            jax.ShapeDtypeStruct((N, D), jnp.float32),
        ],
    )(pos1p, pos2p, feat2, w0cp, w0f2)


_SC_NC = 2
_SC_NS = 16
_NW = _SC_NC * _SC_NS
_BPW = NB // _NW
_CH = 256
_NCH = _BPW // _CH


def _sc_gather(table, idx3):
    mesh = plsc.VectorSubcoreMesh(core_axis_name="c", subcore_axis_name="s")

    @functools.partial(
        pl.kernel,
        mesh=mesh,
        out_type=jax.ShapeDtypeStruct((NB, D), jnp.float32),
        scratch_types=[
            pltpu.VMEM((_CH,), jnp.int32),
            pltpu.VMEM((_CH, D), jnp.float32),
            pltpu.SemaphoreType.DMA,
        ],
    )
    def run(table_hbm, idx_hbm, out_hbm, idx_v, rows_v, sem):
        wid = lax.axis_index("s") * _SC_NC + lax.axis_index("c")
        base = wid * _BPW
        for c in range(_NCH):
            pltpu.sync_copy(idx_hbm.at[wid, c], idx_v)
            pltpu.async_copy(table_hbm.at[idx_v], rows_v, sem).wait()
            pltpu.sync_copy(rows_v, out_hbm.at[pl.ds(base + c * _CH, _CH)])

    return run(table, idx3)


def _mlp0_body(hg_ref, f1_ref, p_ref, w_ref, hout_ref, ssum_ref, ssq_ref):
    i = pl.program_id(0)
    f = lax.dot_general(f1_ref[...].reshape(MB * K, F1), w_ref[...],
                        (((1,), (0,)), ((), ())),
                        preferred_element_type=jnp.float32)
    h = hg_ref[...] + f.reshape(MB, K, D) - p_ref[...]
    hout_ref[...] = h
    h2 = h.reshape(MB * K, D)

    @pl.when(i == 0)
    def _init():
        ssum_ref[...] = jnp.zeros_like(ssum_ref)
        ssq_ref[...] = jnp.zeros_like(ssq_ref)

    ssum_ref[...] += jnp.sum(h2, axis=0, keepdims=True)
    ssq_ref[...] += jnp.sum(h2 * h2, axis=0, keepdims=True)


def _mlp0(hg, f1r, p3, w0f1):
    return pl.pallas_call(
        _mlp0_body,
        grid=(N // MB,),
        in_specs=[
            pl.BlockSpec((MB, K, D), lambda i: (i, 0, 0)),
            pl.BlockSpec((MB, K, F1), lambda i: (i, 0, 0)),
            pl.BlockSpec((MB, 1, D), lambda i: (i, 0, 0)),
            pl.BlockSpec((F1, D), lambda i: (0, 0)),
        ],
        out_specs=[
            pl.BlockSpec((MB, K, D), lambda i: (i, 0, 0)),
            pl.BlockSpec((1, D), lambda i: (0, 0)),
            pl.BlockSpec((1, D), lambda i: (0, 0)),
        ],
        out_shape=[
            jax.ShapeDtypeStruct((N, K, D), jnp.float32),
            jax.ShapeDtypeStruct((1, D), jnp.float32),
            jax.ShapeDtypeStruct((1, D), jnp.float32),
        ],
    )(hg, f1r, p3, w0f1)


def _bn_scale_shift(ssum, ssq, g, b):
    mu = ssum * (1.0 / NB)
    var = ssq * (1.0 / NB) - mu * mu
    inv = lax.rsqrt(var + EPS)
    scale = g * inv
    shift = b - mu * scale
    return scale, shift


def _mlpn_body(h_ref, ssum_in, ssq_in, g_ref, b_ref, w_ref,
               hout_ref, ssum_ref, ssq_ref):
    i = pl.program_id(0)
    scale, shift = _bn_scale_shift(ssum_in[...], ssq_in[...],
                                   g_ref[...], b_ref[...])
    h = h_ref[...] * scale + shift
    a = jnp.where(h >= 0, h, 0.01 * h)
    hn = lax.dot_general(a, w_ref[...], (((1,), (0,)), ((), ())),
                         preferred_element_type=jnp.float32)
    hout_ref[...] = hn

    @pl.when(i == 0)
    def _init():
        ssum_ref[...] = jnp.zeros_like(ssum_ref)
        ssq_ref[...] = jnp.zeros_like(ssq_ref)

    ssum_ref[...] += jnp.sum(hn, axis=0, keepdims=True)
    ssq_ref[...] += jnp.sum(hn * hn, axis=0, keepdims=True)


def _mlpn(h, ssum, ssq, g, b, w):
    return pl.pallas_call(
        _mlpn_body,
        grid=(NB // RB,),
        in_specs=[
            pl.BlockSpec((RB, D), lambda i: (i, 0)),
            pl.BlockSpec((1, D), lambda i: (0, 0)),
            pl.BlockSpec((1, D), lambda i: (0, 0)),
            pl.BlockSpec((1, D), lambda i: (0, 0)),
            pl.BlockSpec((1, D), lambda i: (0, 0)),
            pl.BlockSpec((D, D), lambda i: (0, 0)),
        ],
        out_specs=[
            pl.BlockSpec((RB, D), lambda i: (i, 0)),
            pl.BlockSpec((1, D), lambda i: (0, 0)),
            pl.BlockSpec((1, D), lambda i: (0, 0)),
        ],
        out_shape=[
            jax.ShapeDtypeStruct((NB, D), jnp.float32),
            jax.ShapeDtypeStruct((1, D), jnp.float32),
            jax.ShapeDtypeStruct((1, D), jnp.float32),
        ],
    )(h, ssum, ssq, g, b, w)


def _final_body(h_ref, ssum_in, ssq_in, g_ref, b_ref, out_ref):
    scale, shift = _bn_scale_shift(ssum_in[...], ssq_in[...],
                                   g_ref[...], b_ref[...])
    h = h_ref[...] * scale + shift
    a = jnp.where(h >= 0, h, 0.01 * h)
    out_ref[...] = jnp.max(a, axis=1)


def _final(h3, ssum, ssq, g, b):
    return pl.pallas_call(
        _final_body,
        grid=(N // MB,),
        in_specs=[
            pl.BlockSpec((MB, K, D), lambda i: (i, 0, 0)),
            pl.BlockSpec((1, D), lambda i: (0, 0)),
            pl.BlockSpec((1, D), lambda i: (0, 0)),
            pl.BlockSpec((1, D), lambda i: (0, 0)),
            pl.BlockSpec((1, D), lambda i: (0, 0)),
        ],
        out_specs=pl.BlockSpec((MB, D), lambda i: (i, 0)),
        out_shape=jax.ShapeDtypeStruct((N, D), jnp.float32),
    )(h3, ssum, ssq, g, b)


def kernel(pos1, pos2, feat1, feat2, W0, g0, b0, W1, g1, b1, W2, g2, b2):
    pos1p = jnp.zeros((N, 8), jnp.float32).at[:, :3].set(pos1)
    pos2p = jnp.zeros((N, 8), jnp.float32).at[:, :3].set(pos2)
    pos2t = pos2p.T

    idx = _knn(pos1p, pos2t)

    w0cp = jnp.zeros((8, D), jnp.float32).at[:3].set(W0[:3])
    G, P = _pre(pos1p, pos2p, feat2, w0cp, W0[3:3 + C2])

    idx3 = idx.reshape(_NW, _NCH, _CH)
    hg = _sc_gather(G, idx3)

    h0, s0, q0 = _mlp0(hg.reshape(N, K, D), feat1.reshape(N, K, F1),
                       P.reshape(N, 1, D), W0[3 + C2:])
    h1, s1, q1 = _mlpn(h0.reshape(NB, D), s0, q0,
                       g0.reshape(1, D), b0.reshape(1, D), W1)
    h2, s2, q2 = _mlpn(h1, s1, q1, g1.reshape(1, D), b1.reshape(1, D), W2)
    out = _final(h2.reshape(N, K, D), s2, q2,
                 g2.reshape(1, D), b2.reshape(1, D))
    return out
